# initial kernel scaffold (unmeasured)
import jax
import jax.numpy as jnp
from jax import lax
from jax.experimental import pallas as pl
from jax.experimental.pallas import tpu as pltpu

N_DEV = 4
E_LOCAL = 8
N_EXPERTS = 32
N_TOK = 2048
D_MODEL = 1024


def kernel(x, router_W, route_idx, expert_W):
    def body(x_ref, rw_ref, idx_ref, ew_ref, out_ref,
             comm_ref, wbuf_ref, send_sems, recv_sems, wsems):
        my = lax.axis_index("i")
        left = lax.rem(my + N_DEV - 1, N_DEV)
        right = lax.rem(my + 1, N_DEV)

        barrier_sem = pltpu.get_barrier_semaphore()
        for nbr in (left, right):
            pl.semaphore_signal(
                barrier_sem, inc=1,
                device_id=(nbr,), device_id_type=pl.DeviceIdType.MESH,
            )
        pl.semaphore_wait(barrier_sem, 2)

        xv = x_ref[:, :]
        scores = jnp.dot(xv, rw_ref[:, :], preferred_element_type=jnp.float32)
        m = jnp.max(scores, axis=1, keepdims=True)
        p = jnp.exp(scores - m)
        p = p / jnp.sum(p, axis=1, keepdims=True)
        idx0 = idx_ref[:, 0:1]
        idx1 = idx_ref[:, 1:2]
        cols = lax.broadcasted_iota(jnp.int32, (N_TOK, N_EXPERTS), 1)
        p0 = jnp.sum(jnp.where(cols == idx0, p, 0.0), axis=1, keepdims=True)
        p1 = jnp.sum(jnp.where(cols == idx1, p, 0.0), axis=1, keepdims=True)
        denom = p0 + p1
        w0 = p0 / denom
        w1 = p1 / denom

        def fetch(e, slot):
            return pltpu.make_async_copy(
                ew_ref.at[e], wbuf_ref.at[slot], wsems.at[slot]
            )

        fetch(0, 0).start()
        acc = jnp.zeros((N_TOK, D_MODEL), jnp.float32)
        for e in range(E_LOCAL):
            slot = e % 2
            if e + 1 < E_LOCAL:
                fetch(e + 1, (e + 1) % 2).start()
            fetch(e, slot).wait()
            eg = my * E_LOCAL + e
            g = (jnp.where(idx0 == eg, w0, 0.0)
                 + jnp.where(idx1 == eg, w1, 0.0))
            acc = acc + jnp.dot(
                g * xv, wbuf_ref[slot],
                preferred_element_type=jnp.float32,
            )

        out_ref[:, :] = acc
        comm_ref[0, :, :] = acc

        for h in range(N_DEV - 1):
            send_slot = h % 2
            recv_slot = (h + 1) % 2
            rdma = pltpu.make_async_remote_copy(
                src_ref=comm_ref.at[send_slot],
                dst_ref=comm_ref.at[recv_slot],
                send_sem=send_sems.at[send_slot],
                recv_sem=recv_sems.at[recv_slot],
                device_id=(right,),
                device_id_type=pl.DeviceIdType.MESH,
            )
            rdma.start()
            rdma.wait()
            out_ref[:, :] = out_ref[:, :] + comm_ref[recv_slot, :, :]

    return pl.pallas_call(
        body,
        out_shape=jax.ShapeDtypeStruct((N_TOK, D_MODEL), jnp.float32),
        in_specs=[
            pl.BlockSpec(memory_space=pltpu.VMEM),
            pl.BlockSpec(memory_space=pltpu.VMEM),
            pl.BlockSpec(memory_space=pltpu.VMEM),
            pl.BlockSpec(memory_space=pltpu.ANY),
        ],
        out_specs=pl.BlockSpec(memory_space=pltpu.VMEM),
        scratch_shapes=[
            pltpu.VMEM((2, N_TOK, D_MODEL), jnp.float32),
            pltpu.VMEM((2, D_MODEL, D_MODEL), jnp.float32),
            pltpu.SemaphoreType.DMA((2,)),
            pltpu.SemaphoreType.DMA((2,)),
            pltpu.SemaphoreType.DMA((2,)),
        ],
        compiler_params=pltpu.CompilerParams(collective_id=0),
    )(x, router_W, route_idx, expert_W)


# baseline (device time: 424537 ns/iter reference)
import jax
import jax.numpy as jnp
from jax import lax
from jax.experimental import pallas as pl
from jax.experimental.pallas import tpu as pltpu

N_DEV = 4
E_LOCAL = 8
N_EXPERTS = 32
N_TOK = 2048
D_MODEL = 1024
BLK_M = 512
M_BLKS = N_TOK // BLK_M


def _partial_body(x_ref, rw_ref, idx_ref, ew_ref, out_ref):
    e = pl.program_id(1)
    my = lax.axis_index("i")

    xv = x_ref[:, :]
    scores = jnp.dot(xv, rw_ref[:, :], preferred_element_type=jnp.float32)
    m = jnp.max(scores, axis=1, keepdims=True)
    p = jnp.exp(scores - m)
    p = p / jnp.sum(p, axis=1, keepdims=True)
    idx0 = idx_ref[:, 0:1]
    idx1 = idx_ref[:, 1:2]
    cols = lax.broadcasted_iota(jnp.int32, (BLK_M, N_EXPERTS), 1)
    p0 = jnp.sum(jnp.where(cols == idx0, p, 0.0), axis=1, keepdims=True)
    p1 = jnp.sum(jnp.where(cols == idx1, p, 0.0), axis=1, keepdims=True)
    denom = p0 + p1
    eg = my * E_LOCAL + e
    g = (jnp.where(idx0 == eg, p0 / denom, 0.0)
         + jnp.where(idx1 == eg, p1 / denom, 0.0))

    contrib = jnp.dot(g * xv, ew_ref[0], preferred_element_type=jnp.float32)

    @pl.when(e == 0)
    def _():
        out_ref[:, :] = contrib

    @pl.when(e != 0)
    def _():
        out_ref[:, :] = out_ref[:, :] + contrib


def _moe_partial(x, router_W, route_idx, expert_W):
    return pl.pallas_call(
        _partial_body,
        grid=(M_BLKS, E_LOCAL),
        out_shape=jax.ShapeDtypeStruct((N_TOK, D_MODEL), jnp.float32),
        in_specs=[
            pl.BlockSpec((BLK_M, D_MODEL), lambda b, e: (b, 0)),
            pl.BlockSpec((D_MODEL, N_EXPERTS), lambda b, e: (0, 0)),
            pl.BlockSpec((BLK_M, 2), lambda b, e: (b, 0)),
            pl.BlockSpec((1, D_MODEL, D_MODEL), lambda b, e: (e, 0, 0)),
        ],
        out_specs=pl.BlockSpec((BLK_M, D_MODEL), lambda b, e: (b, 0)),
    )(x, router_W, route_idx, expert_W)


def _allreduce_body(p_ref, out_ref, comm_ref, send_sems, recv_sems):
    my = lax.axis_index("i")
    left = lax.rem(my + N_DEV - 1, N_DEV)
    right = lax.rem(my + 1, N_DEV)

    barrier_sem = pltpu.get_barrier_semaphore()
    for nbr in (left, right):
        pl.semaphore_signal(
            barrier_sem, inc=1,
            device_id=(nbr,), device_id_type=pl.DeviceIdType.MESH,
        )
    pl.semaphore_wait(barrier_sem, 2)

    for c in range(M_BLKS):
        rows = pl.ds(c * BLK_M, BLK_M)
        out_ref[rows, :] = p_ref[rows, :]
        comm_ref[0, rows, :] = p_ref[rows, :]

    for h in range(N_DEV - 1):
        send_slot = h % 2
        recv_slot = (h + 1) % 2
        rdma = pltpu.make_async_remote_copy(
            src_ref=comm_ref.at[send_slot],
            dst_ref=comm_ref.at[recv_slot],
            send_sem=send_sems.at[send_slot],
            recv_sem=recv_sems.at[recv_slot],
            device_id=(right,),
            device_id_type=pl.DeviceIdType.MESH,
        )
        rdma.start()
        rdma.wait()
        for c in range(M_BLKS):
            rows = pl.ds(c * BLK_M, BLK_M)
            out_ref[rows, :] = out_ref[rows, :] + comm_ref[recv_slot, rows, :]


def _ring_allreduce(partial):
    return pl.pallas_call(
        _allreduce_body,
        out_shape=jax.ShapeDtypeStruct((N_TOK, D_MODEL), jnp.float32),
        in_specs=[pl.BlockSpec(memory_space=pltpu.VMEM)],
        out_specs=pl.BlockSpec(memory_space=pltpu.VMEM),
        scratch_shapes=[
            pltpu.VMEM((2, N_TOK, D_MODEL), jnp.float32),
            pltpu.SemaphoreType.DMA((2,)),
            pltpu.SemaphoreType.DMA((2,)),
        ],
        compiler_params=pltpu.CompilerParams(collective_id=0),
    )(partial)


def kernel(x, router_W, route_idx, expert_W):
    partial = _moe_partial(x, router_W, route_idx, expert_W)
    return _ring_allreduce(partial)


# device time: 224600 ns/iter; 1.8902x vs baseline; 1.8902x over previous
import jax
import jax.numpy as jnp
from jax import lax
from jax.experimental import pallas as pl
from jax.experimental.pallas import tpu as pltpu

N_DEV = 4
E_LOCAL = 8
N_EXPERTS = 32
N_TOK = 2048
D_MODEL = 1024
BLK_M = 512
M_BLKS = N_TOK // BLK_M


def _partial_body(x_ref, rw_ref, idx_ref, ew_ref, out_ref):
    e = pl.program_id(1)
    my = lax.axis_index("i")

    xv = x_ref[:, :]
    scores = jnp.dot(xv, rw_ref[:, :], preferred_element_type=jnp.float32)
    m = jnp.max(scores, axis=1, keepdims=True)
    p = jnp.exp(scores - m)
    p = p / jnp.sum(p, axis=1, keepdims=True)
    idx0 = idx_ref[:, 0:1]
    idx1 = idx_ref[:, 1:2]
    cols = lax.broadcasted_iota(jnp.int32, (BLK_M, N_EXPERTS), 1)
    p0 = jnp.sum(jnp.where(cols == idx0, p, 0.0), axis=1, keepdims=True)
    p1 = jnp.sum(jnp.where(cols == idx1, p, 0.0), axis=1, keepdims=True)
    denom = p0 + p1
    eg = my * E_LOCAL + e
    g = (jnp.where(idx0 == eg, p0 / denom, 0.0)
         + jnp.where(idx1 == eg, p1 / denom, 0.0))

    contrib = jnp.dot(g * xv, ew_ref[0], preferred_element_type=jnp.float32)

    @pl.when(e == 0)
    def _():
        out_ref[:, :] = contrib

    @pl.when(e != 0)
    def _():
        out_ref[:, :] = out_ref[:, :] + contrib


def _moe_partial(x, router_W, route_idx, expert_W):
    return pl.pallas_call(
        _partial_body,
        grid=(M_BLKS, E_LOCAL),
        out_shape=jax.ShapeDtypeStruct((N_TOK, D_MODEL), jnp.float32),
        in_specs=[
            pl.BlockSpec((BLK_M, D_MODEL), lambda b, e: (b, 0)),
            pl.BlockSpec((D_MODEL, N_EXPERTS), lambda b, e: (0, 0)),
            pl.BlockSpec((BLK_M, 2), lambda b, e: (b, 0)),
            pl.BlockSpec((1, D_MODEL, D_MODEL), lambda b, e: (e, 0, 0)),
        ],
        out_specs=pl.BlockSpec((BLK_M, D_MODEL), lambda b, e: (b, 0)),
    )(x, router_W, route_idx, expert_W)


BLK_R = N_TOK // N_DEV
HALF_D = D_MODEL // 2


def _allreduce_body(p_ref, out_ref, rs_buf, sbuf,
                    rs_ssem, rs_rsem, ag_ssem, ag_rsem):
    my = lax.axis_index("i")
    left = lax.rem(my + N_DEV - 1, N_DEV)
    right = lax.rem(my + 1, N_DEV)
    nbr = (right, left)

    barrier_sem = pltpu.get_barrier_semaphore()
    for n in (left, right):
        pl.semaphore_signal(
            barrier_sem, inc=1,
            device_id=(n,), device_id_type=pl.DeviceIdType.MESH,
        )
    pl.semaphore_wait(barrier_sem, 2)

    def rows(c):
        return pl.ds(c * BLK_R, BLK_R)

    def cols(d):
        return slice(d * HALF_D, (d + 1) * HALF_D)

    def chunk(expr):
        return lax.rem(expr + 2 * N_DEV, N_DEV)

    for h in range(N_DEV - 1):
        rdmas = []
        for d in range(2):
            send_c = chunk(my - h) if d == 0 else chunk(my + h)
            src = (p_ref.at[rows(send_c), cols(d)] if h == 0
                   else sbuf.at[d, h - 1])
            rdma = pltpu.make_async_remote_copy(
                src_ref=src,
                dst_ref=rs_buf.at[d, h],
                send_sem=rs_ssem.at[d, h],
                recv_sem=rs_rsem.at[d, h],
                device_id=(nbr[d],),
                device_id_type=pl.DeviceIdType.MESH,
            )
            rdma.start()
            rdmas.append(rdma)
        for d in range(2):
            rdmas[d].wait()
            recv_c = chunk(my - 1 - h) if d == 0 else chunk(my + 1 + h)
            acc = rs_buf[d, h] + p_ref[rows(recv_c), cols(d)]
            if h < N_DEV - 2:
                sbuf[d, h] = acc
            else:
                out_ref[rows(recv_c), cols(d)] = acc

    for h in range(N_DEV - 1):
        rdmas = []
        for d in range(2):
            send_c = chunk(my + 1 - h) if d == 0 else chunk(my - 1 + h)
            sl = (rows(send_c), cols(d))
            rdma = pltpu.make_async_remote_copy(
                src_ref=out_ref.at[sl],
                dst_ref=out_ref.at[sl],
                send_sem=ag_ssem.at[d, h],
                recv_sem=ag_rsem.at[d, h],
                device_id=(nbr[d],),
                device_id_type=pl.DeviceIdType.MESH,
            )
            rdma.start()
            rdmas.append(rdma)
        for d in range(2):
            rdmas[d].wait()


def _ring_allreduce(partial):
    return pl.pallas_call(
        _allreduce_body,
        out_shape=jax.ShapeDtypeStruct((N_TOK, D_MODEL), jnp.float32),
        in_specs=[pl.BlockSpec(memory_space=pltpu.VMEM)],
        out_specs=pl.BlockSpec(memory_space=pltpu.VMEM),
        scratch_shapes=[
            pltpu.VMEM((2, N_DEV - 1, BLK_R, HALF_D), jnp.float32),
            pltpu.VMEM((2, N_DEV - 2, BLK_R, HALF_D), jnp.float32),
            pltpu.SemaphoreType.DMA((2, N_DEV - 1)),
            pltpu.SemaphoreType.DMA((2, N_DEV - 1)),
            pltpu.SemaphoreType.DMA((2, N_DEV - 1)),
            pltpu.SemaphoreType.DMA((2, N_DEV - 1)),
        ],
        compiler_params=pltpu.CompilerParams(collective_id=0),
    )(partial)


def kernel(x, router_W, route_idx, expert_W):
    partial = _moe_partial(x, router_W, route_idx, expert_W)
    return _ring_allreduce(partial)


# device time: 209579 ns/iter; 2.0257x vs baseline; 1.0717x over previous
import jax
import jax.numpy as jnp
from jax import lax
from jax.experimental import pallas as pl
from jax.experimental.pallas import tpu as pltpu

N_DEV = 4
E_LOCAL = 8
N_EXPERTS = 32
N_TOK = 2048
D_MODEL = 1024
BLK_M = 512
M_BLKS = N_TOK // BLK_M


def _partial_body(x_ref, rw_ref, idx_ref, ew_ref, out_ref):
    e = pl.program_id(1)
    my = lax.axis_index("i")

    xv = x_ref[:, :]
    scores = jnp.dot(xv, rw_ref[:, :], preferred_element_type=jnp.float32)
    m = jnp.max(scores, axis=1, keepdims=True)
    p = jnp.exp(scores - m)
    p = p / jnp.sum(p, axis=1, keepdims=True)
    idx0 = idx_ref[:, 0:1]
    idx1 = idx_ref[:, 1:2]
    cols = lax.broadcasted_iota(jnp.int32, (BLK_M, N_EXPERTS), 1)
    p0 = jnp.sum(jnp.where(cols == idx0, p, 0.0), axis=1, keepdims=True)
    p1 = jnp.sum(jnp.where(cols == idx1, p, 0.0), axis=1, keepdims=True)
    denom = p0 + p1
    eg = my * E_LOCAL + e
    g = (jnp.where(idx0 == eg, p0 / denom, 0.0)
         + jnp.where(idx1 == eg, p1 / denom, 0.0))

    contrib = jnp.dot(
        (g * xv).astype(jnp.bfloat16),
        ew_ref[0].astype(jnp.bfloat16),
        preferred_element_type=jnp.float32,
    )

    @pl.when(e == 0)
    def _():
        out_ref[:, :] = contrib

    @pl.when(e != 0)
    def _():
        out_ref[:, :] = out_ref[:, :] + contrib


def _moe_partial(x, router_W, route_idx, expert_W):
    return pl.pallas_call(
        _partial_body,
        grid=(M_BLKS, E_LOCAL),
        out_shape=jax.ShapeDtypeStruct((N_TOK, D_MODEL), jnp.float32),
        in_specs=[
            pl.BlockSpec((BLK_M, D_MODEL), lambda b, e: (b, 0)),
            pl.BlockSpec((D_MODEL, N_EXPERTS), lambda b, e: (0, 0)),
            pl.BlockSpec((BLK_M, 2), lambda b, e: (b, 0)),
            pl.BlockSpec((1, D_MODEL, D_MODEL), lambda b, e: (e, 0, 0)),
        ],
        out_specs=pl.BlockSpec((BLK_M, D_MODEL), lambda b, e: (b, 0)),
    )(x, router_W, route_idx, expert_W)


BLK_R = N_TOK // N_DEV
HALF_D = D_MODEL // 2


def _allreduce_body(p_ref, out_ref, rs_buf, sbuf,
                    rs_ssem, rs_rsem, ag_ssem, ag_rsem):
    my = lax.axis_index("i")
    left = lax.rem(my + N_DEV - 1, N_DEV)
    right = lax.rem(my + 1, N_DEV)
    nbr = (right, left)

    barrier_sem = pltpu.get_barrier_semaphore()
    for n in (left, right):
        pl.semaphore_signal(
            barrier_sem, inc=1,
            device_id=(n,), device_id_type=pl.DeviceIdType.MESH,
        )
    pl.semaphore_wait(barrier_sem, 2)

    def rows(c):
        return pl.ds(c * BLK_R, BLK_R)

    def cols(d):
        return slice(d * HALF_D, (d + 1) * HALF_D)

    def chunk(expr):
        return lax.rem(expr + 2 * N_DEV, N_DEV)

    for h in range(N_DEV - 1):
        rdmas = []
        for d in range(2):
            send_c = chunk(my - h) if d == 0 else chunk(my + h)
            src = (p_ref.at[rows(send_c), cols(d)] if h == 0
                   else sbuf.at[d, h - 1])
            rdma = pltpu.make_async_remote_copy(
                src_ref=src,
                dst_ref=rs_buf.at[d, h],
                send_sem=rs_ssem.at[d, h],
                recv_sem=rs_rsem.at[d, h],
                device_id=(nbr[d],),
                device_id_type=pl.DeviceIdType.MESH,
            )
            rdma.start()
            rdmas.append(rdma)
        for d in range(2):
            rdmas[d].wait()
            recv_c = chunk(my - 1 - h) if d == 0 else chunk(my + 1 + h)
            acc = rs_buf[d, h] + p_ref[rows(recv_c), cols(d)]
            if h < N_DEV - 2:
                sbuf[d, h] = acc
            else:
                out_ref[rows(recv_c), cols(d)] = acc

    for h in range(N_DEV - 1):
        rdmas = []
        for d in range(2):
            send_c = chunk(my + 1 - h) if d == 0 else chunk(my - 1 + h)
            sl = (rows(send_c), cols(d))
            rdma = pltpu.make_async_remote_copy(
                src_ref=out_ref.at[sl],
                dst_ref=out_ref.at[sl],
                send_sem=ag_ssem.at[d, h],
                recv_sem=ag_rsem.at[d, h],
                device_id=(nbr[d],),
                device_id_type=pl.DeviceIdType.MESH,
            )
            rdma.start()
            rdmas.append(rdma)
        for d in range(2):
            rdmas[d].wait()


def _ring_allreduce(partial):
    return pl.pallas_call(
        _allreduce_body,
        out_shape=jax.ShapeDtypeStruct((N_TOK, D_MODEL), jnp.float32),
        in_specs=[pl.BlockSpec(memory_space=pltpu.VMEM)],
        out_specs=pl.BlockSpec(memory_space=pltpu.VMEM),
        scratch_shapes=[
            pltpu.VMEM((2, N_DEV - 1, BLK_R, HALF_D), jnp.float32),
            pltpu.VMEM((2, N_DEV - 2, BLK_R, HALF_D), jnp.float32),
            pltpu.SemaphoreType.DMA((2, N_DEV - 1)),
            pltpu.SemaphoreType.DMA((2, N_DEV - 1)),
            pltpu.SemaphoreType.DMA((2, N_DEV - 1)),
            pltpu.SemaphoreType.DMA((2, N_DEV - 1)),
        ],
        compiler_params=pltpu.CompilerParams(collective_id=0),
    )(partial)


def kernel(x, router_W, route_idx, expert_W):
    partial = _moe_partial(x, router_W, route_idx, expert_W)
    return _ring_allreduce(partial)


# device time: 139395 ns/iter; 3.0456x vs baseline; 1.5035x over previous
import jax
import jax.numpy as jnp
from jax import lax
from jax.experimental import pallas as pl
from jax.experimental.pallas import tpu as pltpu

N_DEV = 4
E_LOCAL = 8
N_EXPERTS = 32
N_TOK = 2048
D_MODEL = 1024
BLK_M = 512
M_BLKS = N_TOK // BLK_M


def _partial_body(x_ref, rw_ref, idx_ref, ew_ref, out_ref):
    e = pl.program_id(0)
    my = lax.axis_index("i")

    scores = jnp.dot(x_ref[:, :], rw_ref[:, :],
                     preferred_element_type=jnp.float32)
    m = jnp.max(scores, axis=1, keepdims=True)
    p = jnp.exp(scores - m)
    p = p / jnp.sum(p, axis=1, keepdims=True)
    idx0 = idx_ref[:, 0:1]
    idx1 = idx_ref[:, 1:2]
    cols = lax.broadcasted_iota(jnp.int32, (N_TOK, N_EXPERTS), 1)
    p0 = jnp.sum(jnp.where(cols == idx0, p, 0.0), axis=1, keepdims=True)
    p1 = jnp.sum(jnp.where(cols == idx1, p, 0.0), axis=1, keepdims=True)
    denom = p0 + p1
    eg = my * E_LOCAL + e
    g = (jnp.where(idx0 == eg, p0 / denom, 0.0)
         + jnp.where(idx1 == eg, p1 / denom, 0.0))

    w_bf = ew_ref[0].astype(jnp.bfloat16)
    for rb in range(M_BLKS):
        rows = slice(rb * BLK_M, (rb + 1) * BLK_M)
        gx = (g[rows, :] * x_ref[rows, :]).astype(jnp.bfloat16)
        contrib = jnp.dot(gx, w_bf, preferred_element_type=jnp.float32)

        @pl.when(e == 0)
        def _():
            out_ref[rows, :] = contrib

        @pl.when(e != 0)
        def _():
            out_ref[rows, :] = out_ref[rows, :] + contrib


def _moe_partial(x, router_W, route_idx, expert_W):
    return pl.pallas_call(
        _partial_body,
        grid=(E_LOCAL,),
        out_shape=jax.ShapeDtypeStruct((N_TOK, D_MODEL), jnp.float32),
        in_specs=[
            pl.BlockSpec((N_TOK, D_MODEL), lambda e: (0, 0)),
            pl.BlockSpec((D_MODEL, N_EXPERTS), lambda e: (0, 0)),
            pl.BlockSpec((N_TOK, 2), lambda e: (0, 0)),
            pl.BlockSpec((1, D_MODEL, D_MODEL), lambda e: (e, 0, 0)),
        ],
        out_specs=pl.BlockSpec((N_TOK, D_MODEL), lambda e: (0, 0)),
    )(x, router_W, route_idx, expert_W)


BLK_R = N_TOK // N_DEV
HALF_D = D_MODEL // 2


def _allreduce_body(p_ref, out_ref, rs_buf, sbuf, ag_buf,
                    rs_ssem, rs_rsem, ag_ssem, ag_rsem):
    my = lax.axis_index("i")
    left = lax.rem(my + N_DEV - 1, N_DEV)
    right = lax.rem(my + 1, N_DEV)
    nbr = (right, left)

    barrier_sem = pltpu.get_barrier_semaphore()
    for n in (left, right):
        pl.semaphore_signal(
            barrier_sem, inc=1,
            device_id=(n,), device_id_type=pl.DeviceIdType.MESH,
        )
    pl.semaphore_wait(barrier_sem, 2)

    def rows(c):
        return pl.ds(c * BLK_R, BLK_R)

    def cols(d):
        return slice(d * HALF_D, (d + 1) * HALF_D)

    def chunk(expr):
        return lax.rem(expr + 2 * N_DEV, N_DEV)

    for d in range(2):
        sbuf[d, 0] = p_ref[rows(my), cols(d)].astype(jnp.bfloat16)

    for h in range(N_DEV - 1):
        rdmas = []
        for d in range(2):
            rdma = pltpu.make_async_remote_copy(
                src_ref=sbuf.at[d, h],
                dst_ref=rs_buf.at[d, h],
                send_sem=rs_ssem.at[d, h],
                recv_sem=rs_rsem.at[d, h],
                device_id=(nbr[d],),
                device_id_type=pl.DeviceIdType.MESH,
            )
            rdma.start()
            rdmas.append(rdma)
        for d in range(2):
            rdmas[d].wait()
            recv_c = chunk(my - 1 - h) if d == 0 else chunk(my + 1 + h)
            acc = (rs_buf[d, h].astype(jnp.float32)
                   + p_ref[rows(recv_c), cols(d)])
            if h < N_DEV - 2:
                sbuf[d, h + 1] = acc.astype(jnp.bfloat16)
            else:
                out_ref[rows(recv_c), cols(d)] = acc
                sbuf[d, 0] = acc.astype(jnp.bfloat16)

    for h in range(N_DEV - 1):
        rdmas = []
        for d in range(2):
            src = sbuf.at[d, 0] if h == 0 else ag_buf.at[d, h - 1]
            rdma = pltpu.make_async_remote_copy(
                src_ref=src,
                dst_ref=ag_buf.at[d, h],
                send_sem=ag_ssem.at[d, h],
                recv_sem=ag_rsem.at[d, h],
                device_id=(nbr[d],),
                device_id_type=pl.DeviceIdType.MESH,
            )
            rdma.start()
            rdmas.append(rdma)
        for d in range(2):
            rdmas[d].wait()
            recv_c = chunk(my - h) if d == 0 else chunk(my + h)
            out_ref[rows(recv_c), cols(d)] = ag_buf[d, h].astype(jnp.float32)


def _ring_allreduce(partial):
    return pl.pallas_call(
        _allreduce_body,
        out_shape=jax.ShapeDtypeStruct((N_TOK, D_MODEL), jnp.float32),
        in_specs=[pl.BlockSpec(memory_space=pltpu.VMEM)],
        out_specs=pl.BlockSpec(memory_space=pltpu.VMEM),
        scratch_shapes=[
            pltpu.VMEM((2, N_DEV - 1, BLK_R, HALF_D), jnp.bfloat16),
            pltpu.VMEM((2, N_DEV - 1, BLK_R, HALF_D), jnp.bfloat16),
            pltpu.VMEM((2, N_DEV - 1, BLK_R, HALF_D), jnp.bfloat16),
            pltpu.SemaphoreType.DMA((2, N_DEV - 1)),
            pltpu.SemaphoreType.DMA((2, N_DEV - 1)),
            pltpu.SemaphoreType.DMA((2, N_DEV - 1)),
            pltpu.SemaphoreType.DMA((2, N_DEV - 1)),
        ],
        compiler_params=pltpu.CompilerParams(collective_id=0),
    )(partial)


def kernel(x, router_W, route_idx, expert_W):
    partial = _moe_partial(x, router_W, route_idx, expert_W)
    return _ring_allreduce(partial)


# device time: 111558 ns/iter; 3.8055x vs baseline; 1.2495x over previous
import jax
import jax.numpy as jnp
from jax import lax
from jax.experimental import pallas as pl
from jax.experimental.pallas import tpu as pltpu

N_DEV = 4
E_LOCAL = 8
N_EXPERTS = 32
N_TOK = 2048
D_MODEL = 1024
BLK_M = 512
M_BLKS = N_TOK // BLK_M


def _partial_body(x_ref, rw_ref, idx_ref, ew_ref, out_ref):
    e = pl.program_id(0)
    my = lax.axis_index("i")

    scores = jnp.dot(x_ref[:, :], rw_ref[:, :],
                     preferred_element_type=jnp.float32)
    m = jnp.max(scores, axis=1, keepdims=True)
    p = jnp.exp(scores - m)
    p = p / jnp.sum(p, axis=1, keepdims=True)
    idx0 = idx_ref[:, 0:1]
    idx1 = idx_ref[:, 1:2]
    cols = lax.broadcasted_iota(jnp.int32, (N_TOK, N_EXPERTS), 1)
    p0 = jnp.sum(jnp.where(cols == idx0, p, 0.0), axis=1, keepdims=True)
    p1 = jnp.sum(jnp.where(cols == idx1, p, 0.0), axis=1, keepdims=True)
    denom = p0 + p1
    eg = my * E_LOCAL + e
    g = (jnp.where(idx0 == eg, p0 / denom, 0.0)
         + jnp.where(idx1 == eg, p1 / denom, 0.0))

    w_bf = ew_ref[0].astype(jnp.bfloat16)
    for rb in range(M_BLKS):
        rows = slice(rb * BLK_M, (rb + 1) * BLK_M)
        gx = (g[rows, :] * x_ref[rows, :]).astype(jnp.bfloat16)
        contrib = jnp.dot(gx, w_bf, preferred_element_type=jnp.float32)

        @pl.when(e == 0)
        def _():
            out_ref[rows, :] = contrib

        @pl.when(e != 0)
        def _():
            out_ref[rows, :] = out_ref[rows, :] + contrib


def _moe_partial(x, router_W, route_idx, expert_W):
    return pl.pallas_call(
        _partial_body,
        grid=(E_LOCAL,),
        out_shape=jax.ShapeDtypeStruct((N_TOK, D_MODEL), jnp.float32),
        in_specs=[
            pl.BlockSpec((N_TOK, D_MODEL), lambda e: (0, 0)),
            pl.BlockSpec((D_MODEL, N_EXPERTS), lambda e: (0, 0)),
            pl.BlockSpec((N_TOK, 2), lambda e: (0, 0)),
            pl.BlockSpec((1, D_MODEL, D_MODEL), lambda e: (e, 0, 0)),
        ],
        out_specs=pl.BlockSpec((N_TOK, D_MODEL), lambda e: (0, 0)),
    )(x, router_W, route_idx, expert_W)


BLK_R = N_TOK // N_DEV
HALF_D = D_MODEL // 2


def _allreduce_body(p_ref, out_ref, rs_buf, sbuf, ag_buf,
                    rs_ssem, rs_rsem, ag_ssem, ag_rsem):
    my = lax.axis_index("i")
    left = lax.rem(my + N_DEV - 1, N_DEV)
    right = lax.rem(my + 1, N_DEV)
    nbr = (right, left)

    barrier_sem = pltpu.get_barrier_semaphore()
    for n in (left, right):
        pl.semaphore_signal(
            barrier_sem, inc=1,
            device_id=(n,), device_id_type=pl.DeviceIdType.MESH,
        )
    pl.semaphore_wait(barrier_sem, 2)

    def rows(c):
        return pl.ds(c * BLK_R, BLK_R)

    def cols(d):
        return slice(d * HALF_D, (d + 1) * HALF_D)

    def chunk(expr):
        return lax.rem(expr + 2 * N_DEV, N_DEV)

    for d in range(2):
        sbuf[d, 0] = p_ref[rows(my), cols(d)].astype(jnp.bfloat16)

    for h in range(N_DEV - 1):
        rdmas = []
        for d in range(2):
            rdma = pltpu.make_async_remote_copy(
                src_ref=sbuf.at[d, h],
                dst_ref=rs_buf.at[d, h],
                send_sem=rs_ssem.at[d, h],
                recv_sem=rs_rsem.at[d, h],
                device_id=(nbr[d],),
                device_id_type=pl.DeviceIdType.MESH,
            )
            rdma.start()
            rdmas.append(rdma)
        for d in range(2):
            rdmas[d].wait()
            recv_c = chunk(my - 1 - h) if d == 0 else chunk(my + 1 + h)
            acc = (rs_buf[d, h].astype(jnp.float32)
                   + p_ref[rows(recv_c), cols(d)])
            if h < N_DEV - 2:
                sbuf[d, h + 1] = acc.astype(jnp.bfloat16)
            else:
                out_ref[rows(recv_c), cols(d)] = acc
                sbuf[d, 0] = acc.astype(jnp.bfloat16)

    for h in range(N_DEV - 1):
        rdmas = []
        for d in range(2):
            src = sbuf.at[d, 0] if h == 0 else ag_buf.at[d, h - 1]
            rdma = pltpu.make_async_remote_copy(
                src_ref=src,
                dst_ref=ag_buf.at[d, h],
                send_sem=ag_ssem.at[d, h],
                recv_sem=ag_rsem.at[d, h],
                device_id=(nbr[d],),
                device_id_type=pl.DeviceIdType.MESH,
            )
            rdma.start()
            rdmas.append(rdma)
        for d in range(2):
            rdmas[d].wait()
            recv_c = chunk(my - h) if d == 0 else chunk(my + h)
            out_ref[rows(recv_c), cols(d)] = ag_buf[d, h].astype(jnp.float32)


def _ring_allreduce(partial):
    return pl.pallas_call(
        _allreduce_body,
        out_shape=jax.ShapeDtypeStruct((N_TOK, D_MODEL), jnp.float32),
        in_specs=[pl.BlockSpec(memory_space=pltpu.VMEM)],
        out_specs=pl.BlockSpec(memory_space=pltpu.VMEM),
        scratch_shapes=[
            pltpu.VMEM((2, N_DEV - 1, BLK_R, HALF_D), jnp.bfloat16),
            pltpu.VMEM((2, N_DEV - 1, BLK_R, HALF_D), jnp.bfloat16),
            pltpu.VMEM((2, N_DEV - 1, BLK_R, HALF_D), jnp.bfloat16),
            pltpu.SemaphoreType.DMA((2, N_DEV - 1)),
            pltpu.SemaphoreType.DMA((2, N_DEV - 1)),
            pltpu.SemaphoreType.DMA((2, N_DEV - 1)),
            pltpu.SemaphoreType.DMA((2, N_DEV - 1)),
        ],
        compiler_params=pltpu.CompilerParams(collective_id=0),
    )(partial)


def _fused_body(x_ref, rw_ref, idx_ref, ew_hbm, out_ref,
                ge_ref, wbuf, wsems,
                rs_buf, sbuf, ag_buf,
                rs_ssem, rs_rsem, ag_ssem, ag_rsem):
    my = lax.axis_index("i")
    left = lax.rem(my + N_DEV - 1, N_DEV)
    right = lax.rem(my + 1, N_DEV)
    nbr = (right, left)

    def fetch(e, slot):
        return pltpu.make_async_copy(ew_hbm.at[e], wbuf.at[slot],
                                     wsems.at[slot])

    fetch(0, 0).start()

    scores = jnp.dot(x_ref[:, :], rw_ref[:, :],
                     preferred_element_type=jnp.float32)
    m = jnp.max(scores, axis=1, keepdims=True)
    p = jnp.exp(scores - m)
    p = p / jnp.sum(p, axis=1, keepdims=True)
    idx0 = idx_ref[:, 0:1]
    idx1 = idx_ref[:, 1:2]
    colsv = lax.broadcasted_iota(jnp.int32, (N_TOK, N_EXPERTS), 1)
    p0 = jnp.sum(jnp.where(colsv == idx0, p, 0.0), axis=1, keepdims=True)
    p1 = jnp.sum(jnp.where(colsv == idx1, p, 0.0), axis=1, keepdims=True)
    denom = p0 + p1
    w0 = p0 / denom
    w1 = p1 / denom
    egrid = my * E_LOCAL + lax.broadcasted_iota(
        jnp.int32, (N_TOK, E_LOCAL), 1)
    ge_ref[:, :] = (jnp.where(idx0 == egrid, w0, 0.0)
                    + jnp.where(idx1 == egrid, w1, 0.0))

    for e in range(E_LOCAL):
        slot = e % 2
        fetch(e, slot).wait()
        if e + 1 < E_LOCAL:
            fetch(e + 1, (e + 1) % 2).start()
        w_bf = wbuf[slot].astype(jnp.bfloat16)
        for rb in range(M_BLKS):
            rows = slice(rb * BLK_M, (rb + 1) * BLK_M)
            gx = (ge_ref[rows, e:e + 1]
                  * x_ref[rows, :]).astype(jnp.bfloat16)
            contrib = jnp.dot(gx, w_bf, preferred_element_type=jnp.float32)
            if e == 0:
                out_ref[rows, :] = contrib
            else:
                out_ref[rows, :] = out_ref[rows, :] + contrib

    barrier_sem = pltpu.get_barrier_semaphore()
    for n in (left, right):
        pl.semaphore_signal(
            barrier_sem, inc=1,
            device_id=(n,), device_id_type=pl.DeviceIdType.MESH,
        )
    pl.semaphore_wait(barrier_sem, 2)

    def rows_of(c):
        return pl.ds(c * BLK_R, BLK_R)

    def cols_of(d):
        return slice(d * HALF_D, (d + 1) * HALF_D)

    def chunk(expr):
        return lax.rem(expr + 2 * N_DEV, N_DEV)

    for d in range(2):
        sbuf[d, 0] = out_ref[rows_of(my), cols_of(d)].astype(jnp.bfloat16)

    for h in range(N_DEV - 1):
        rdmas = []
        for d in range(2):
            rdma = pltpu.make_async_remote_copy(
                src_ref=sbuf.at[d, h],
                dst_ref=rs_buf.at[d, h],
                send_sem=rs_ssem.at[d, h],
                recv_sem=rs_rsem.at[d, h],
                device_id=(nbr[d],),
                device_id_type=pl.DeviceIdType.MESH,
            )
            rdma.start()
            rdmas.append(rdma)
        for d in range(2):
            rdmas[d].wait()
            recv_c = chunk(my - 1 - h) if d == 0 else chunk(my + 1 + h)
            acc = (rs_buf[d, h].astype(jnp.float32)
                   + out_ref[rows_of(recv_c), cols_of(d)])
            if h < N_DEV - 2:
                sbuf[d, h + 1] = acc.astype(jnp.bfloat16)
            else:
                out_ref[rows_of(recv_c), cols_of(d)] = acc
                sbuf[d, 0] = acc.astype(jnp.bfloat16)

    for h in range(N_DEV - 1):
        rdmas = []
        for d in range(2):
            src = sbuf.at[d, 0] if h == 0 else ag_buf.at[d, h - 1]
            rdma = pltpu.make_async_remote_copy(
                src_ref=src,
                dst_ref=ag_buf.at[d, h],
                send_sem=ag_ssem.at[d, h],
                recv_sem=ag_rsem.at[d, h],
                device_id=(nbr[d],),
                device_id_type=pl.DeviceIdType.MESH,
            )
            rdma.start()
            rdmas.append(rdma)
        for d in range(2):
            rdmas[d].wait()
            recv_c = chunk(my - h) if d == 0 else chunk(my + h)
            out_ref[rows_of(recv_c), cols_of(d)] = (
                ag_buf[d, h].astype(jnp.float32))


def kernel(x, router_W, route_idx, expert_W):
    return pl.pallas_call(
        _fused_body,
        out_shape=jax.ShapeDtypeStruct((N_TOK, D_MODEL), jnp.float32),
        in_specs=[
            pl.BlockSpec(memory_space=pltpu.VMEM),
            pl.BlockSpec(memory_space=pltpu.VMEM),
            pl.BlockSpec(memory_space=pltpu.VMEM),
            pl.BlockSpec(memory_space=pl.ANY),
        ],
        out_specs=pl.BlockSpec(memory_space=pltpu.VMEM),
        scratch_shapes=[
            pltpu.VMEM((N_TOK, E_LOCAL), jnp.float32),
            pltpu.VMEM((2, D_MODEL, D_MODEL), jnp.float32),
            pltpu.SemaphoreType.DMA((2,)),
            pltpu.VMEM((2, N_DEV - 1, BLK_R, HALF_D), jnp.bfloat16),
            pltpu.VMEM((2, N_DEV - 1, BLK_R, HALF_D), jnp.bfloat16),
            pltpu.VMEM((2, N_DEV - 1, BLK_R, HALF_D), jnp.bfloat16),
            pltpu.SemaphoreType.DMA((2, N_DEV - 1)),
            pltpu.SemaphoreType.DMA((2, N_DEV - 1)),
            pltpu.SemaphoreType.DMA((2, N_DEV - 1)),
            pltpu.SemaphoreType.DMA((2, N_DEV - 1)),
        ],
        compiler_params=pltpu.CompilerParams(collective_id=0),
    )(x, router_W, route_idx, expert_W)


# device time: 107463 ns/iter; 3.9505x vs baseline; 1.0381x over previous
import jax
import jax.numpy as jnp
from jax import lax
from jax.experimental import pallas as pl
from jax.experimental.pallas import tpu as pltpu

N_DEV = 4
E_LOCAL = 8
N_EXPERTS = 32
N_TOK = 2048
D_MODEL = 1024
BLK_M = 512
M_BLKS = N_TOK // BLK_M

N_GRP = 2
GRP_D = D_MODEL // N_GRP
BLK_R = N_TOK // N_DEV
QCOL = GRP_D // 2
N_HOP = N_DEV - 1


def _fused_body(x_ref, rw_ref, idx_ref, ew_hbm, out_ref,
                ge_ref, wbuf, wsems,
                rs_buf, sbuf, ag_buf,
                rs_ssem, rs_rsem, ag_ssem, ag_rsem):
    my = lax.axis_index("i")
    left = lax.rem(my + N_DEV - 1, N_DEV)
    right = lax.rem(my + 1, N_DEV)
    nbr = (right, left)

    def fetch(k, slot):
        g, e = divmod(k, E_LOCAL)
        return pltpu.make_async_copy(
            ew_hbm.at[e, :, pl.ds(g * GRP_D, GRP_D)],
            wbuf.at[slot], wsems.at[slot])

    fetch(0, 0).start()
    fetch(1, 1).start()

    scores = jnp.dot(x_ref[:, :], rw_ref[:, :],
                     preferred_element_type=jnp.float32)
    m = jnp.max(scores, axis=1, keepdims=True)
    p = jnp.exp(scores - m)
    p = p / jnp.sum(p, axis=1, keepdims=True)
    idx0 = idx_ref[:, 0:1]
    idx1 = idx_ref[:, 1:2]
    colsv = lax.broadcasted_iota(jnp.int32, (N_TOK, N_EXPERTS), 1)
    p0 = jnp.sum(jnp.where(colsv == idx0, p, 0.0), axis=1, keepdims=True)
    p1 = jnp.sum(jnp.where(colsv == idx1, p, 0.0), axis=1, keepdims=True)
    denom = p0 + p1
    egrid = my * E_LOCAL + lax.broadcasted_iota(
        jnp.int32, (N_TOK, E_LOCAL), 1)
    ge_ref[:, :] = (jnp.where(idx0 == egrid, p0 / denom, 0.0)
                    + jnp.where(idx1 == egrid, p1 / denom, 0.0))

    def compute_expert(g, e):
        k = g * E_LOCAL + e
        slot = k % 2
        fetch(k, slot).wait()
        w_bf = wbuf[slot].astype(jnp.bfloat16)
        gcols = slice(g * GRP_D, (g + 1) * GRP_D)
        for rb in range(M_BLKS):
            rows = slice(rb * BLK_M, (rb + 1) * BLK_M)
            gx = (ge_ref[rows, e:e + 1]
                  * x_ref[rows, :]).astype(jnp.bfloat16)
            contrib = jnp.dot(gx, w_bf, preferred_element_type=jnp.float32)
            if e == 0:
                out_ref[rows, gcols] = contrib
            else:
                out_ref[rows, gcols] = out_ref[rows, gcols] + contrib
        if k + 2 < N_GRP * E_LOCAL:
            fetch(k + 2, slot).start()

    def rows_of(c):
        return pl.ds(c * BLK_R, BLK_R)

    def cols_of(g, d):
        base = g * GRP_D + d * QCOL
        return slice(base, base + QCOL)

    def chunk(expr):
        return lax.rem(expr + 2 * N_DEV, N_DEV)

    def rs_rdma(g, d, h):
        return pltpu.make_async_remote_copy(
            src_ref=sbuf.at[g, d, h],
            dst_ref=rs_buf.at[g, d, h],
            send_sem=rs_ssem.at[g, d, h],
            recv_sem=rs_rsem.at[g, d, h],
            device_id=(nbr[d],),
            device_id_type=pl.DeviceIdType.MESH,
        )

    def ag_rdma(g, d, h):
        return pltpu.make_async_remote_copy(
            src_ref=sbuf.at[g, d, 0] if h == 0 else ag_buf.at[g, d, h - 1],
            dst_ref=ag_buf.at[g, d, h],
            send_sem=ag_ssem.at[g, d, h],
            recv_sem=ag_rsem.at[g, d, h],
            device_id=(nbr[d],),
            device_id_type=pl.DeviceIdType.MESH,
        )

    def comm_start(g):
        for d in range(2):
            sbuf[g, d, 0] = out_ref[rows_of(my), cols_of(g, d)].astype(
                jnp.bfloat16)
        for d in range(2):
            rs_rdma(g, d, 0).start()

    def comm_step(g, s):
        if s < N_HOP:
            h = s
            for d in range(2):
                rs_rdma(g, d, h).wait()
                recv_c = chunk(my - 1 - h) if d == 0 else chunk(my + 1 + h)
                acc = (rs_buf[g, d, h].astype(jnp.float32)
                       + out_ref[rows_of(recv_c), cols_of(g, d)])
                if h < N_HOP - 1:
                    sbuf[g, d, h + 1] = acc.astype(jnp.bfloat16)
                else:
                    out_ref[rows_of(recv_c), cols_of(g, d)] = acc
                    sbuf[g, d, 0] = acc.astype(jnp.bfloat16)
            for d in range(2):
                if h < N_HOP - 1:
                    rs_rdma(g, d, h + 1).start()
                else:
                    ag_rdma(g, d, 0).start()
        else:
            h = s - N_HOP
            for d in range(2):
                ag_rdma(g, d, h).wait()
                recv_c = chunk(my - h) if d == 0 else chunk(my + h)
                out_ref[rows_of(recv_c), cols_of(g, d)] = (
                    ag_buf[g, d, h].astype(jnp.float32))
            if h < N_HOP - 1:
                for d in range(2):
                    ag_rdma(g, d, h + 1).start()

    for e in range(E_LOCAL):
        compute_expert(0, e)

    barrier_sem = pltpu.get_barrier_semaphore()
    for n in (left, right):
        pl.semaphore_signal(
            barrier_sem, inc=1,
            device_id=(n,), device_id_type=pl.DeviceIdType.MESH,
        )
    pl.semaphore_wait(barrier_sem, 2)

    comm_start(0)

    for e in range(E_LOCAL):
        compute_expert(1, e)
        if 1 <= e <= 6:
            comm_step(0, e - 1)

    comm_start(1)
    for s in range(2 * N_HOP):
        comm_step(1, s)


def kernel(x, router_W, route_idx, expert_W):
    return pl.pallas_call(
        _fused_body,
        out_shape=jax.ShapeDtypeStruct((N_TOK, D_MODEL), jnp.float32),
        in_specs=[
            pl.BlockSpec(memory_space=pltpu.VMEM),
            pl.BlockSpec(memory_space=pltpu.VMEM),
            pl.BlockSpec(memory_space=pltpu.VMEM),
            pl.BlockSpec(memory_space=pl.ANY),
        ],
        out_specs=pl.BlockSpec(memory_space=pltpu.VMEM),
        scratch_shapes=[
            pltpu.VMEM((N_TOK, E_LOCAL), jnp.float32),
            pltpu.VMEM((2, D_MODEL, GRP_D), jnp.float32),
            pltpu.SemaphoreType.DMA((2,)),
            pltpu.VMEM((N_GRP, 2, N_HOP, BLK_R, QCOL), jnp.bfloat16),
            pltpu.VMEM((N_GRP, 2, N_HOP, BLK_R, QCOL), jnp.bfloat16),
            pltpu.VMEM((N_GRP, 2, N_HOP, BLK_R, QCOL), jnp.bfloat16),
            pltpu.SemaphoreType.DMA((N_GRP, 2, N_HOP)),
            pltpu.SemaphoreType.DMA((N_GRP, 2, N_HOP)),
            pltpu.SemaphoreType.DMA((N_GRP, 2, N_HOP)),
            pltpu.SemaphoreType.DMA((N_GRP, 2, N_HOP)),
        ],
        compiler_params=pltpu.CompilerParams(collective_id=0),
    )(x, router_W, route_idx, expert_W)


# device time: 107335 ns/iter; 3.9553x vs baseline; 1.0012x over previous
import jax
import jax.numpy as jnp
from jax import lax
from jax.experimental import pallas as pl
from jax.experimental.pallas import tpu as pltpu

N_DEV = 4
E_LOCAL = 8
N_EXPERTS = 32
N_TOK = 2048
D_MODEL = 1024
BLK_M = 512
M_BLKS = N_TOK // BLK_M

N_GRP = 2
GRP_D = D_MODEL // N_GRP
BLK_R = N_TOK // N_DEV
QCOL = GRP_D // 2
N_HOP = N_DEV - 1


def _fused_body(x_ref, rw_ref, idx_ref, ew_hbm, out_ref,
                ge_ref, xbf_ref, wbuf, wsems,
                rs_buf, sbuf, ag_buf,
                rs_ssem, rs_rsem, ag_ssem, ag_rsem):
    my = lax.axis_index("i")
    left = lax.rem(my + N_DEV - 1, N_DEV)
    right = lax.rem(my + 1, N_DEV)
    nbr = (right, left)

    def fetch(k, slot):
        g, e = divmod(k, E_LOCAL)
        return pltpu.make_async_copy(
            ew_hbm.at[e, :, pl.ds(g * GRP_D, GRP_D)],
            wbuf.at[slot], wsems.at[slot])

    fetch(0, 0).start()
    fetch(1, 1).start()

    scores = jnp.dot(x_ref[:, :], rw_ref[:, :],
                     preferred_element_type=jnp.float32)
    m = jnp.max(scores, axis=1, keepdims=True)
    p = jnp.exp(scores - m)
    p = p / jnp.sum(p, axis=1, keepdims=True)
    idx0 = idx_ref[:, 0:1]
    idx1 = idx_ref[:, 1:2]
    colsv = lax.broadcasted_iota(jnp.int32, (N_TOK, N_EXPERTS), 1)
    p0 = jnp.sum(jnp.where(colsv == idx0, p, 0.0), axis=1, keepdims=True)
    p1 = jnp.sum(jnp.where(colsv == idx1, p, 0.0), axis=1, keepdims=True)
    denom = p0 + p1
    egrid = my * E_LOCAL + lax.broadcasted_iota(
        jnp.int32, (N_TOK, E_LOCAL), 1)
    ge_ref[:, :] = (jnp.where(idx0 == egrid, p0 / denom, 0.0)
                    + jnp.where(idx1 == egrid, p1 / denom, 0.0)
                    ).astype(jnp.bfloat16)
    for rb in range(M_BLKS):
        rows = slice(rb * BLK_M, (rb + 1) * BLK_M)
        xbf_ref[rows, :] = x_ref[rows, :].astype(jnp.bfloat16)

    def dots(g, e, w_bf):
        gcols = slice(g * GRP_D, (g + 1) * GRP_D)
        for rb in range(M_BLKS):
            rows = slice(rb * BLK_M, (rb + 1) * BLK_M)
            gx = ge_ref[rows, e:e + 1] * xbf_ref[rows, :]
            contrib = jnp.dot(gx, w_bf, preferred_element_type=jnp.float32)
            if e == 0:
                out_ref[rows, gcols] = contrib
            else:
                out_ref[rows, gcols] = out_ref[rows, gcols] + contrib

    def compute_expert(g, e):
        k = g * E_LOCAL + e
        slot = k % 2
        fetch(k, slot).wait()
        w_bf = wbuf[slot].astype(jnp.bfloat16)
        dots(g, e, w_bf)
        if k + 2 < N_GRP * E_LOCAL:
            fetch(k + 2, slot).start()

    def rows_of(c):
        return pl.ds(c * BLK_R, BLK_R)

    def cols_of(g, d):
        base = g * GRP_D + d * QCOL
        return slice(base, base + QCOL)

    def chunk(expr):
        return lax.rem(expr + 2 * N_DEV, N_DEV)

    def rs_rdma(d, h):
        return pltpu.make_async_remote_copy(
            src_ref=sbuf.at[d, h],
            dst_ref=rs_buf.at[d, h],
            send_sem=rs_ssem.at[d, h],
            recv_sem=rs_rsem.at[d, h],
            device_id=(nbr[d],),
            device_id_type=pl.DeviceIdType.MESH,
        )

    def ag_rdma(d, h):
        return pltpu.make_async_remote_copy(
            src_ref=sbuf.at[d, 0] if h == 0 else ag_buf.at[d, h - 1],
            dst_ref=ag_buf.at[d, h],
            send_sem=ag_ssem.at[d, h],
            recv_sem=ag_rsem.at[d, h],
            device_id=(nbr[d],),
            device_id_type=pl.DeviceIdType.MESH,
        )

    def comm_start(g):
        for d in range(2):
            sbuf[d, 0] = out_ref[rows_of(my), cols_of(g, d)].astype(
                jnp.bfloat16)
        for d in range(2):
            rs_rdma(d, 0).start()

    def comm_step(g, s):
        if s < N_HOP:
            h = s
            for d in range(2):
                rs_rdma(d, h).wait()
                recv_c = chunk(my - 1 - h) if d == 0 else chunk(my + 1 + h)
                acc = (rs_buf[d, h].astype(jnp.float32)
                       + out_ref[rows_of(recv_c), cols_of(g, d)])
                if h < N_HOP - 1:
                    sbuf[d, h + 1] = acc.astype(jnp.bfloat16)
                else:
                    out_ref[rows_of(recv_c), cols_of(g, d)] = acc
                    sbuf[d, 0] = acc.astype(jnp.bfloat16)
            for d in range(2):
                if h < N_HOP - 1:
                    rs_rdma(d, h + 1).start()
                else:
                    ag_rdma(d, 0).start()
        else:
            h = s - N_HOP
            for d in range(2):
                ag_rdma(d, h).wait()
                recv_c = chunk(my - h) if d == 0 else chunk(my + h)
                out_ref[rows_of(recv_c), cols_of(g, d)] = (
                    ag_buf[d, h].astype(jnp.float32))
            if h < N_HOP - 1:
                for d in range(2):
                    ag_rdma(d, h + 1).start()

    for e in range(E_LOCAL):
        compute_expert(0, e)

    barrier_sem = pltpu.get_barrier_semaphore()
    for n in (left, right):
        pl.semaphore_signal(
            barrier_sem, inc=1,
            device_id=(n,), device_id_type=pl.DeviceIdType.MESH,
        )
    pl.semaphore_wait(barrier_sem, 2)

    comm_start(0)

    for e in range(E_LOCAL):
        compute_expert(1, e)
        if 1 <= e <= 6:
            comm_step(0, e - 1)

    comm_start(1)
    for s in range(2 * N_HOP):
        comm_step(1, s)


def kernel(x, router_W, route_idx, expert_W):
    return pl.pallas_call(
        _fused_body,
        out_shape=jax.ShapeDtypeStruct((N_TOK, D_MODEL), jnp.float32),
        in_specs=[
            pl.BlockSpec(memory_space=pltpu.VMEM),
            pl.BlockSpec(memory_space=pltpu.VMEM),
            pl.BlockSpec(memory_space=pltpu.VMEM),
            pl.BlockSpec(memory_space=pl.ANY),
        ],
        out_specs=pl.BlockSpec(memory_space=pltpu.VMEM),
        scratch_shapes=[
            pltpu.VMEM((N_TOK, E_LOCAL), jnp.bfloat16),
            pltpu.VMEM((N_TOK, D_MODEL), jnp.bfloat16),
            pltpu.VMEM((2, D_MODEL, GRP_D), jnp.float32),
            pltpu.SemaphoreType.DMA((2,)),
            pltpu.VMEM((2, N_HOP, BLK_R, QCOL), jnp.bfloat16),
            pltpu.VMEM((2, N_HOP, BLK_R, QCOL), jnp.bfloat16),
            pltpu.VMEM((2, N_HOP, BLK_R, QCOL), jnp.bfloat16),
            pltpu.SemaphoreType.DMA((2, N_HOP)),
            pltpu.SemaphoreType.DMA((2, N_HOP)),
            pltpu.SemaphoreType.DMA((2, N_HOP)),
            pltpu.SemaphoreType.DMA((2, N_HOP)),
        ],
        compiler_params=pltpu.CompilerParams(collective_id=0),
    )(x, router_W, route_idx, expert_W)
